# Initial kernel scaffold; baseline (speedup 1.0000x reference)
import jax
import jax.numpy as jnp
from jax.experimental import pallas as pl


def kernel(x, edge_index, W1, b1, W2, b2):
    raise NotImplementedError("write your pallas kernel here")


# speedup vs baseline: 27.7508x; 27.7508x over previous
import functools

import jax
import jax.numpy as jnp
from jax import lax
from jax.experimental import pallas as pl
from jax.experimental.pallas import tpu as pltpu
from jax.experimental.pallas import tpu_sc as plsc

_NC = 2
_NS = 16
_NW = _NC * _NS
_IW = 128


def _make_sc_pass(n_acc, d, e_rows, do_gather):
    rpt = e_rows // _NW
    zr = n_acc // _NS
    mesh = plsc.VectorSubcoreMesh(core_axis_name="c", subcore_axis_name="s",
                                  num_cores=_NC, num_subcores=_NS)

    scratch = [
        pltpu.VMEM_SHARED((n_acc, d), jnp.float32),
        pltpu.VMEM((rpt, _IW), jnp.int32),
    ]
    if do_gather:
        scratch += [
            pltpu.VMEM((rpt, _IW), jnp.int32),
            pltpu.VMEM((_IW, d), jnp.float32),
            pltpu.SemaphoreType.DMA,
        ]
    else:
        scratch += [pltpu.VMEM((_IW, d), jnp.float32)]

    def body(*refs):
        if do_gather:
            (src2d, dst2d, feat, zrows, out,
             acc, didx, sidx, rows, sem) = refs
        else:
            (dst2d, ones_hbm, zrows, out,
             acc, didx, rows) = refs
        c = lax.axis_index("c")
        s = lax.axis_index("s")
        w = c * _NS + s
        pltpu.sync_copy(zrows, acc.at[pl.ds(s * zr, zr)])
        pltpu.sync_copy(dst2d.at[pl.ds(w * rpt, rpt)], didx)
        if do_gather:
            pltpu.sync_copy(src2d.at[pl.ds(w * rpt, rpt)], sidx)
        else:
            pltpu.sync_copy(ones_hbm, rows)
        plsc.subcore_barrier()

        def step(j, carry):
            if do_gather:
                pltpu.async_copy(feat.at[sidx.at[j]], rows, sem).wait()
            pltpu.sync_copy(rows, acc.at[didx.at[j]], add=True)
            return carry

        lax.fori_loop(0, rpt, step, 0)
        plsc.subcore_barrier()
        pltpu.sync_copy(acc.at[pl.ds(s * zr, zr)],
                        out.at[pl.ds(c * n_acc + s * zr, zr)])

    return pl.kernel(
        body,
        out_type=jax.ShapeDtypeStruct((2 * n_acc, d), jnp.float32),
        mesh=mesh,
        scratch_types=scratch,
        compiler_params=pltpu.CompilerParams(use_tc_tiling_on_sc=False),
    )


def _mm1_body(x_ref, w_ref, o_ref):
    o_ref[...] = jnp.dot(x_ref[...], w_ref[...],
                         preferred_element_type=jnp.float32)


def _norm_body(d0_ref, d1_ref, h_ref, dis_ref, hs_ref):
    deg = d0_ref[...] + d1_ref[...] + 1.0
    dis = lax.rsqrt(deg)
    dis_ref[...] = dis
    hs_ref[...] = dis * h_ref[...]


def _post1_body(a0_ref, a1_ref, dis_ref, hs_ref, b_ref, o_ref):
    dis = dis_ref[...]
    out1 = dis * (a0_ref[...] + a1_ref[...] + hs_ref[...]) + b_ref[...]
    o_ref[...] = dis * jnp.maximum(out1, 0.0)


def _post2_body(a0_ref, a1_ref, dis_ref, rs_ref, w_ref, b_ref, o_ref):
    z = dis_ref[...] * (a0_ref[...] + a1_ref[...] + rs_ref[...])
    logits = jnp.dot(z, w_ref[...],
                     preferred_element_type=jnp.float32) + b_ref[...]
    m = jnp.max(logits, axis=1, keepdims=True)
    lse = jnp.log(jnp.sum(jnp.exp(logits - m), axis=1, keepdims=True)) + m
    o_ref[...] = logits - lse


def _row_call(body, n, bn, in_dims, out_dims):
    grid = (n // bn,)

    def spec(dcols):
        if dcols is None:
            return None
        return pl.BlockSpec((bn, dcols), lambda i: (i, 0))

    in_specs = []
    for dc, full in in_dims:
        if full is None:
            in_specs.append(pl.BlockSpec((bn, dc), lambda i: (i, 0)))
        else:
            in_specs.append(pl.BlockSpec(full, lambda i: (0, 0)))
    out_specs = [pl.BlockSpec((bn, dc), lambda i: (i, 0)) for dc in out_dims]
    out_shape = [jax.ShapeDtypeStruct((n, dc), jnp.float32) for dc in out_dims]
    if len(out_specs) == 1:
        out_specs, out_shape = out_specs[0], out_shape[0]
    return pl.pallas_call(
        body,
        grid=grid,
        in_specs=in_specs,
        out_specs=out_specs,
        out_shape=out_shape,
    )


def kernel(x, edge_index, W1, b1, W2, b2):
    n, d_in = x.shape
    e = edge_index.shape[1]
    d_hid = W1.shape[1]
    d_out = W2.shape[1]
    bn = 1000

    chunk = _IW * _NW * 8
    e_pad = -(-e // chunk) * chunk
    e_rows = e_pad // _IW
    n_acc = -(-n // 128) * 128

    src = edge_index[0]
    dst = edge_index[1]
    filler = jnp.full((e_pad - e,), n, dtype=jnp.int32)
    src2d = jnp.concatenate([src, filler]).reshape(e_rows, _IW)
    dst2d = jnp.concatenate([dst, filler]).reshape(e_rows, _IW)

    zrows = jnp.zeros((n_acc // _NS, d_hid), dtype=jnp.float32)
    ones = jnp.ones((_IW, d_hid), dtype=jnp.float32)
    fpad = jnp.zeros((n_acc - n, d_hid), dtype=jnp.float32)

    deg_pass = _make_sc_pass(n_acc, d_hid, e_rows, do_gather=False)
    agg_pass = _make_sc_pass(n_acc, d_hid, e_rows, do_gather=True)

    degp = deg_pass(dst2d, ones, zrows)
    h = _row_call(_mm1_body, n, bn,
                  [(d_in, None), (None, (d_in, d_hid))], [d_hid])(x, W1)

    dis, hs = _row_call(_norm_body, n, bn,
                        [(d_hid, None)] * 3, [d_hid, d_hid])(
        degp[:n], degp[n_acc:n_acc + n], h)

    agg1 = agg_pass(src2d, dst2d, jnp.concatenate([hs, fpad]), zrows)
    rs = _row_call(_post1_body, n, bn,
                   [(d_hid, None)] * 4 + [(None, (1, d_hid))], [d_hid])(
        agg1[:n], agg1[n_acc:n_acc + n], dis, hs, b1.reshape(1, d_hid))

    agg2 = agg_pass(src2d, dst2d, jnp.concatenate([rs, fpad]), zrows)
    out = _row_call(_post2_body, n, bn,
                    [(d_hid, None)] * 4 +
                    [(None, (d_hid, d_out)), (None, (1, d_out))], [d_out])(
        agg2[:n], agg2[n_acc:n_acc + n], dis, rs, W2, b2.reshape(1, d_out))
    return out


# speedup vs baseline: 35.5763x; 1.2820x over previous
import functools

import jax
import jax.numpy as jnp
from jax import lax
from jax.experimental import pallas as pl
from jax.experimental.pallas import tpu as pltpu
from jax.experimental.pallas import tpu_sc as plsc

_NC = 2
_NS = 16
_NW = _NC * _NS
_IW = 128


def _make_sc_pass(n_acc, d, e_rows, do_gather):
    rpt = e_rows // _NW
    zr = n_acc // _NS
    mesh = plsc.VectorSubcoreMesh(core_axis_name="c", subcore_axis_name="s",
                                  num_cores=_NC, num_subcores=_NS)

    nb = 4
    kf = 8
    scratch = [
        pltpu.VMEM_SHARED((n_acc, d), jnp.float32),
        pltpu.VMEM((rpt, _IW), jnp.int32),
    ]
    if do_gather:
        scratch += [
            pltpu.VMEM((rpt, _IW), jnp.int32),
            pltpu.VMEM((nb, _IW, d), jnp.float32),
        ] + [pltpu.SemaphoreType.DMA] * nb
    else:
        scratch += [pltpu.VMEM((_IW, d), jnp.float32),
                    pltpu.SemaphoreType.DMA]

    def body(*refs):
        if do_gather:
            (src2d, dst2d, feat, zrows, out,
             acc, didx, sidx, rows, *sems) = refs
        else:
            (dst2d, ones_hbm, zrows, out,
             acc, didx, rows, sem) = refs
        c = lax.axis_index("c")
        s = lax.axis_index("s")
        w = c * _NS + s
        pltpu.sync_copy(zrows, acc.at[pl.ds(s * zr, zr)])
        pltpu.sync_copy(dst2d.at[pl.ds(w * rpt, rpt)], didx)
        if do_gather:
            pltpu.sync_copy(src2d.at[pl.ds(w * rpt, rpt)], sidx)
        else:
            pltpu.sync_copy(ones_hbm, rows)
        plsc.subcore_barrier()

        if do_gather:
            for b in range(nb):
                pltpu.async_copy(feat.at[sidx.at[b]], rows.at[b], sems[b])

            def group(g, carry):
                for b in range(nb):
                    j = g * nb + b
                    pltpu.make_async_copy(feat.at[sidx.at[j]],
                                          rows.at[b], sems[b]).wait()
                    pltpu.sync_copy(rows.at[b], acc.at[didx.at[j]], add=True)
                    jn = j + nb

                    @pl.when(jn < rpt)
                    def _():
                        pltpu.async_copy(feat.at[sidx.at[jn]],
                                         rows.at[b], sems[b])
                return carry

            lax.fori_loop(0, rpt // nb, group, 0)
        else:
            def group(g, carry):
                for b in range(kf):
                    pltpu.async_copy(rows, acc.at[didx.at[g * kf + b]], sem,
                                     add=True)
                for b in range(kf):
                    pltpu.make_async_copy(rows, acc.at[didx.at[g * kf + b]],
                                          sem).wait()
                return carry

            lax.fori_loop(0, rpt // kf, group, 0)
        plsc.subcore_barrier()
        pltpu.sync_copy(acc.at[pl.ds(s * zr, zr)],
                        out.at[pl.ds(c * n_acc + s * zr, zr)])

    return pl.kernel(
        body,
        out_type=jax.ShapeDtypeStruct((2 * n_acc, d), jnp.float32),
        mesh=mesh,
        scratch_types=scratch,
        compiler_params=pltpu.CompilerParams(use_tc_tiling_on_sc=False),
    )


def _mm1_body(x_ref, w_ref, o_ref):
    o_ref[...] = jnp.dot(x_ref[...], w_ref[...],
                         preferred_element_type=jnp.float32)


def _norm_body(d0_ref, d1_ref, h_ref, dis_ref, hs_ref):
    deg = d0_ref[...] + d1_ref[...] + 1.0
    dis = lax.rsqrt(deg)
    dis_ref[...] = dis
    hs_ref[...] = dis * h_ref[...]


def _post1_body(a0_ref, a1_ref, dis_ref, hs_ref, b_ref, o_ref):
    dis = dis_ref[...]
    out1 = dis * (a0_ref[...] + a1_ref[...] + hs_ref[...]) + b_ref[...]
    o_ref[...] = dis * jnp.maximum(out1, 0.0)


def _post2_body(a0_ref, a1_ref, dis_ref, rs_ref, w_ref, b_ref, o_ref):
    z = dis_ref[...] * (a0_ref[...] + a1_ref[...] + rs_ref[...])
    logits = jnp.dot(z, w_ref[...],
                     preferred_element_type=jnp.float32) + b_ref[...]
    m = jnp.max(logits, axis=1, keepdims=True)
    lse = jnp.log(jnp.sum(jnp.exp(logits - m), axis=1, keepdims=True)) + m
    o_ref[...] = logits - lse


def _row_call(body, n, bn, in_dims, out_dims):
    grid = (n // bn,)

    def spec(dcols):
        if dcols is None:
            return None
        return pl.BlockSpec((bn, dcols), lambda i: (i, 0))

    in_specs = []
    for dc, full in in_dims:
        if full is None:
            in_specs.append(pl.BlockSpec((bn, dc), lambda i: (i, 0)))
        else:
            in_specs.append(pl.BlockSpec(full, lambda i: (0, 0)))
    out_specs = [pl.BlockSpec((bn, dc), lambda i: (i, 0)) for dc in out_dims]
    out_shape = [jax.ShapeDtypeStruct((n, dc), jnp.float32) for dc in out_dims]
    if len(out_specs) == 1:
        out_specs, out_shape = out_specs[0], out_shape[0]
    return pl.pallas_call(
        body,
        grid=grid,
        in_specs=in_specs,
        out_specs=out_specs,
        out_shape=out_shape,
    )


def kernel(x, edge_index, W1, b1, W2, b2):
    n, d_in = x.shape
    e = edge_index.shape[1]
    d_hid = W1.shape[1]
    d_out = W2.shape[1]
    bn = 1000

    chunk = _IW * _NW * 8
    e_pad = -(-e // chunk) * chunk
    e_rows = e_pad // _IW
    n_acc = -(-n // 128) * 128

    src = edge_index[0]
    dst = edge_index[1]
    src2d = jnp.concatenate(
        [src, jnp.zeros((e_pad - e,), dtype=jnp.int32)]).reshape(e_rows, _IW)
    dst2d = jnp.concatenate(
        [dst, jnp.full((e_pad - e,), n, dtype=jnp.int32)]).reshape(e_rows, _IW)

    zrows = jnp.zeros((n_acc // _NS, d_hid), dtype=jnp.float32)
    ones = jnp.ones((_IW, d_hid), dtype=jnp.float32)

    deg_pass = _make_sc_pass(n_acc, d_hid, e_rows, do_gather=False)
    agg_pass = _make_sc_pass(n_acc, d_hid, e_rows, do_gather=True)

    degp = deg_pass(dst2d, ones, zrows)
    h = _row_call(_mm1_body, n, bn,
                  [(d_in, None), (None, (d_in, d_hid))], [d_hid])(x, W1)

    dis, hs = _row_call(_norm_body, n, bn,
                        [(d_hid, None)] * 3, [d_hid, d_hid])(
        degp[:n], degp[n_acc:n_acc + n], h)

    agg1 = agg_pass(src2d, dst2d, hs, zrows)
    rs = _row_call(_post1_body, n, bn,
                   [(d_hid, None)] * 4 + [(None, (1, d_hid))], [d_hid])(
        agg1[:n], agg1[n_acc:n_acc + n], dis, hs, b1.reshape(1, d_hid))

    agg2 = agg_pass(src2d, dst2d, rs, zrows)
    out = _row_call(_post2_body, n, bn,
                    [(d_hid, None)] * 4 +
                    [(None, (d_hid, d_out)), (None, (1, d_out))], [d_out])(
        agg2[:n], agg2[n_acc:n_acc + n], dis, rs, W2, b2.reshape(1, d_out))
    return out
